# baseline (speedup 1.0000x reference)
import functools

import jax
import jax.numpy as jnp
from jax import lax
from jax.experimental import pallas as pl
from jax.experimental.pallas import tpu as pltpu
from jax.experimental.pallas import tpu_sc as plsc

MAXN = 200000
NEWB = 4096
RES = MAXN - NEWB
KEY = 20000

NC = 2
NS = 16
NW = NC * NS
CH = 640
SUBG = 5
BSUB = CH // SUBG
PAD = NW * CH
TRASH = PAD


def _sc_sample(feat, new_feat, reserved, samp2d):
    mesh = plsc.VectorSubcoreMesh(core_axis_name="c", subcore_axis_name="s")

    @functools.partial(
        pl.kernel,
        mesh=mesh,
        out_type=jax.ShapeDtypeStruct((PAD + 8, 256), jnp.float32),
        scratch_types=[
            pltpu.VMEM((CH,), jnp.int32),
            pltpu.VMEM((SUBG, BSUB), jnp.int32),
            pltpu.VMEM((SUBG, BSUB), jnp.int32),
            pltpu.VMEM((SUBG, BSUB), jnp.int32),
            pltpu.VMEM((SUBG, BSUB), jnp.int32),
            pltpu.VMEM((SUBG, BSUB), jnp.int32),
            pltpu.VMEM((BSUB, 256), jnp.float32),
            pltpu.VMEM((BSUB, 256), jnp.float32),
            pltpu.SemaphoreType.DMA,
            pltpu.SemaphoreType.DMA,
            pltpu.SemaphoreType.DMA,
        ],
    )
    def k(feat_h, new_h, res_h, samp_h, out_h,
          s_b, sc_b, r_b, ia_b, ib_b, pos_b, buf_a, buf_b,
          sem_r, sem_a, sem_b):
        wid = lax.axis_index("s") * NC + lax.axis_index("c")
        base = wid * CH
        pltpu.sync_copy(samp_h.at[pl.ds(wid * CH, CH)], s_b)
        for j in range(CH // 16):
            g, col = j // (BSUB // 16), (j % (BSUB // 16)) * 16
            s = s_b[pl.ds(j * 16, 16)]
            sc_b[g, pl.ds(col, 16)] = jnp.minimum(s, RES - 1)
        copies = [
            pltpu.async_copy(res_h.at[sc_b.at[g]], r_b.at[g], sem_r)
            for g in range(SUBG)
        ]
        for c in copies:
            c.wait()
        iota = lax.iota(jnp.int32, 16)
        for j in range(CH // 16):
            g, col = j // (BSUB // 16), (j % (BSUB // 16)) * 16
            s = s_b[pl.ds(j * 16, 16)]
            r = r_b[g, pl.ds(col, 16)]
            m = s < RES
            ia_b[g, pl.ds(col, 16)] = jnp.where(m, r, 0)
            ib_b[g, pl.ds(col, 16)] = jnp.where(m, 0, s - RES)
            pos_b[g, pl.ds(col, 16)] = jnp.where(m, TRASH, base + j * 16 + iota)
        for g in range(SUBG):
            ca = pltpu.async_copy(feat_h.at[ia_b.at[g]], buf_a, sem_a)
            cb = pltpu.async_copy(new_h.at[ib_b.at[g]], buf_b, sem_b)
            ca.wait()
            pltpu.sync_copy(buf_a, out_h.at[pl.ds(base + g * BSUB, BSUB)])
            cb.wait()
            cs = pltpu.async_copy(buf_b, out_h.at[pos_b.at[g]], sem_b)
            cs.wait()

    return k(feat, new_feat, reserved, samp2d)


def kernel(feat, new_feat, reserved_ind, sampled_ind):
    pad = jnp.zeros((PAD - KEY,), dtype=sampled_ind.dtype)
    samp1d = jnp.concatenate([sampled_ind, pad])
    out = _sc_sample(feat, new_feat, reserved_ind, samp1d)
    return out[:KEY]
